# Initial kernel scaffold; baseline (speedup 1.0000x reference)
import jax
import jax.numpy as jnp
from jax.experimental import pallas as pl


def kernel(input):
    raise NotImplementedError("write your pallas kernel here")


# speedup vs baseline: 1.8807x; 1.8807x over previous
import functools

import jax
import jax.numpy as jnp
from jax import lax
from jax.experimental import pallas as pl
from jax.experimental.pallas import tpu as pltpu
from jax.experimental.pallas import tpu_sc as plsc

B = 128
N = 32768
LANES = 16
CHUNKS = N // LANES
NUM_WORKERS = 32
ROWS_PER_WORKER = B // NUM_WORKERS
N_BISECT = 24

_mesh = plsc.VectorSubcoreMesh(core_axis_name="c", subcore_axis_name="s")


@functools.partial(
    pl.kernel,
    mesh=_mesh,
    out_type=jax.ShapeDtypeStruct((B * N,), jnp.float32),
    scratch_types=[pltpu.VMEM((N,), jnp.float32)],
    compiler_params=pltpu.CompilerParams(needs_layout_passes=False),
)
def _sparsemax_sc(x_hbm, out_hbm, buf):
    cid = lax.axis_index("c")
    sid = lax.axis_index("s")
    wid = sid * 2 + cid

    def row_body(r, carry):
        row = wid * ROWS_PER_WORKER + r
        base = row * N
        pltpu.sync_copy(x_hbm.at[pl.ds(base, N)], buf)

        def max_body(i, m):
            return jnp.maximum(m, buf[pl.ds(i * LANES, LANES)])

        mvec = lax.fori_loop(1, CHUNKS, max_body, buf[pl.ds(0, LANES)])
        xmax = jnp.max(mvec)

        def bis_body(t, lohi):
            lo, hi = lohi
            mid = 0.5 * (lo + hi)

            def sum_body(i, acc):
                v = buf[pl.ds(i * LANES, LANES)]
                return acc + jnp.maximum(v - mid, 0.0)

            acc = lax.fori_loop(
                0, CHUNKS, sum_body, jnp.zeros((LANES,), jnp.float32)
            )
            s = jnp.sum(acc)
            gt = s > 1.0
            return jnp.where(gt, mid, lo), jnp.where(gt, hi, mid)

        lo, hi = lax.fori_loop(0, N_BISECT, bis_body, (xmax - 1.0, xmax))

        def ks_body(i, kc):
            ka, sa = kc
            v = buf[pl.ds(i * LANES, LANES)]
            g = v > lo
            return (
                ka + jnp.where(g, 1.0, 0.0),
                sa + jnp.where(g, v, 0.0),
            )

        zero = jnp.zeros((LANES,), jnp.float32)
        ka, sa = lax.fori_loop(0, CHUNKS, ks_body, (zero, zero))
        k = jnp.sum(ka)
        s = jnp.sum(sa)
        tau_v = (jnp.full((LANES,), s) - 1.0) / jnp.full((LANES,), k)

        def out_body(i, c):
            v = buf[pl.ds(i * LANES, LANES)]
            buf[pl.ds(i * LANES, LANES)] = jnp.maximum(v - tau_v, 0.0)
            return c

        lax.fori_loop(0, CHUNKS, out_body, 0)
        pltpu.sync_copy(buf, out_hbm.at[pl.ds(base, N)])
        return carry

    lax.fori_loop(0, ROWS_PER_WORKER, row_body, 0)


def kernel(input):
    out = _sparsemax_sc(input.reshape(-1))
    return out.reshape(input.shape)


# speedup vs baseline: 14.2486x; 7.5764x over previous
import functools

import jax
import jax.numpy as jnp
from jax import lax
from jax.experimental import pallas as pl
from jax.experimental.pallas import tpu as pltpu
from jax.experimental.pallas import tpu_sc as plsc

B = 128
N = 32768
LANES = 16
CHUNKS = N // LANES
NUM_WORKERS = 32
ROWS_PER_WORKER = B // NUM_WORKERS
N_BISECT = 24
ACCS = 8

_mesh = plsc.VectorSubcoreMesh(core_axis_name="c", subcore_axis_name="s")


@functools.partial(
    pl.kernel,
    mesh=_mesh,
    out_type=jax.ShapeDtypeStruct((B * N,), jnp.float32),
    scratch_types=[
        pltpu.VMEM((N,), jnp.float32),
        pltpu.VMEM((N + LANES,), jnp.float32),
    ],
    compiler_params=pltpu.CompilerParams(needs_layout_passes=False),
)
def _sparsemax_sc(x_hbm, out_hbm, buf, cbuf):
    cid = lax.axis_index("c")
    sid = lax.axis_index("s")
    wid = sid * 2 + cid

    def row_body(r, carry):
        row = wid * ROWS_PER_WORKER + r
        base = row * N
        pltpu.sync_copy(x_hbm.at[pl.ds(base, N)], buf)

        def max_body(i, accs):
            return tuple(
                jnp.maximum(a, buf[pl.ds((i * ACCS + j) * LANES, LANES)])
                for j, a in enumerate(accs)
            )

        init = tuple(
            buf[pl.ds(j * LANES, LANES)] for j in range(ACCS)
        )
        accs = lax.fori_loop(1, CHUNKS // ACCS, max_body, init)
        mvec = accs[0]
        for a in accs[1:]:
            mvec = jnp.maximum(mvec, a)
        xmax = jnp.max(mvec)
        thresh = xmax - 1.0

        def compact_body(i, off):
            for j in range(ACCS):
                v = buf[pl.ds((i * ACCS + j) * LANES, LANES)]
                g = v > thresh
                plsc.store_compressed(cbuf.at[pl.ds(off, LANES)], v, mask=g)
                off = off + plsc.all_reduce_population_count(g)[0]
            return off

        m = lax.fori_loop(0, CHUNKS // ACCS, compact_body, jnp.int32(0))
        cbuf[pl.ds(m, LANES)] = jnp.full((LANES,), thresh)
        nch = jnp.right_shift(m, 4) + 1

        def bis_body(t, lohi):
            lo, hi = lohi
            mid = 0.5 * (lo + hi)

            def sum_body(i, acc):
                v = cbuf[pl.ds(i * LANES, LANES)]
                return acc + jnp.maximum(v - mid, 0.0)

            acc = lax.fori_loop(
                0, nch, sum_body, jnp.zeros((LANES,), jnp.float32)
            )
            s = jnp.sum(acc)
            gt = s > 1.0
            return jnp.where(gt, mid, lo), jnp.where(gt, hi, mid)

        lo, hi = lax.fori_loop(0, N_BISECT, bis_body, (thresh, xmax))

        def ks_body(i, kc):
            ka, sa = kc
            v = cbuf[pl.ds(i * LANES, LANES)]
            g = v > lo
            return (
                ka + jnp.where(g, 1.0, 0.0),
                sa + jnp.where(g, v, 0.0),
            )

        zero = jnp.zeros((LANES,), jnp.float32)
        ka, sa = lax.fori_loop(0, nch, ks_body, (zero, zero))
        k = jnp.maximum(jnp.sum(ka), 1.0)
        s = jnp.sum(sa)
        tau_v = (jnp.full((LANES,), s) - 1.0) / jnp.full((LANES,), k)

        @plsc.parallel_loop(0, CHUNKS, unroll=ACCS)
        def out_body(i):
            v = buf[pl.ds(i * LANES, LANES)]
            buf[pl.ds(i * LANES, LANES)] = jnp.maximum(v - tau_v, 0.0)

        pltpu.sync_copy(buf, out_hbm.at[pl.ds(base, N)])
        return carry

    lax.fori_loop(0, ROWS_PER_WORKER, row_body, 0)


def kernel(input):
    out = _sparsemax_sc(input.reshape(-1))
    return out.reshape(input.shape)


# speedup vs baseline: 42.9143x; 3.0118x over previous
import functools

import jax
import jax.numpy as jnp
from jax import lax
from jax.experimental import pallas as pl
from jax.experimental.pallas import tpu as pltpu
from jax.experimental.pallas import tpu_sc as plsc

B = 128
N = 32768
LANES = 16
CHUNKS = N // LANES
NUM_WORKERS = 32
ROWS_PER_WORKER = B // NUM_WORKERS
N_BISECT = 24
ACCS = 8

_mesh = plsc.VectorSubcoreMesh(core_axis_name="c", subcore_axis_name="s")


def _row_sparsemax(buf, cbuf):
    def max_body(i, accs):
        return tuple(
            jnp.maximum(a, buf[pl.ds((i * ACCS + j) * LANES, LANES)])
            for j, a in enumerate(accs)
        )

    init = tuple(buf[pl.ds(j * LANES, LANES)] for j in range(ACCS))
    accs = lax.fori_loop(1, CHUNKS // ACCS, max_body, init)
    mvec = accs[0]
    for a in accs[1:]:
        mvec = jnp.maximum(mvec, a)
    xmax = jnp.max(mvec)
    thresh = xmax - 1.0
    thresh_v = jnp.full((LANES,), thresh)

    @plsc.parallel_loop(
        0, CHUNKS, unroll=ACCS, carry=jnp.zeros((LANES,), jnp.int32)
    )
    def compact_body(i, off_v):
        v = buf[pl.ds(i * LANES, LANES)]
        g = v > thresh_v
        ps = plsc.cumsum(jnp.where(g, 1, 0).astype(jnp.int32))
        plsc.store_scatter(cbuf, [off_v + ps - 1], v, mask=g)
        return off_v + plsc.all_reduce_population_count(g)

    m = compact_body[0]
    cbuf[pl.ds(m, LANES)] = thresh_v
    nch = jnp.right_shift(m, 4) + 1

    def bis_body(t, lohi):
        lo, hi = lohi
        mid = 0.5 * (lo + hi)

        def sum_body(i, acc):
            v = cbuf[pl.ds(i * LANES, LANES)]
            return acc + jnp.maximum(v - mid, 0.0)

        acc = lax.fori_loop(0, nch, sum_body, jnp.zeros((LANES,), jnp.float32))
        s = jnp.sum(acc)
        gt = s > 1.0
        return jnp.where(gt, mid, lo), jnp.where(gt, hi, mid)

    lo, hi = lax.fori_loop(0, N_BISECT, bis_body, (thresh, xmax))

    def ks_body(i, kc):
        ka, sa = kc
        v = cbuf[pl.ds(i * LANES, LANES)]
        g = v > lo
        return (
            ka + jnp.where(g, 1.0, 0.0),
            sa + jnp.where(g, v, 0.0),
        )

    zero = jnp.zeros((LANES,), jnp.float32)
    ka, sa = lax.fori_loop(0, nch, ks_body, (zero, zero))
    k = jnp.maximum(jnp.sum(ka), 1.0)
    s = jnp.sum(sa)
    tau_v = (jnp.full((LANES,), s) - 1.0) / jnp.full((LANES,), k)

    @plsc.parallel_loop(0, CHUNKS, unroll=ACCS)
    def out_body(i):
        v = buf[pl.ds(i * LANES, LANES)]
        buf[pl.ds(i * LANES, LANES)] = jnp.maximum(v - tau_v, 0.0)


@functools.partial(
    pl.kernel,
    mesh=_mesh,
    out_type=jax.ShapeDtypeStruct((B, N), jnp.float32),
    scratch_types=[
        pltpu.VMEM((N,), jnp.float32),
        pltpu.VMEM((N,), jnp.float32),
        pltpu.VMEM((N + LANES,), jnp.float32),
        pltpu.SemaphoreType.DMA,
        pltpu.SemaphoreType.DMA,
        pltpu.SemaphoreType.DMA,
        pltpu.SemaphoreType.DMA,
    ],
    compiler_params=pltpu.CompilerParams(needs_layout_passes=False),
)
def _sparsemax_sc(x_hbm, out_hbm, buf0, buf1, cbuf, gsem0, gsem1, ssem0, ssem1):
    cid = lax.axis_index("c")
    sid = lax.axis_index("s")
    wid = sid * 2 + cid
    row0 = wid * ROWS_PER_WORKER

    bufs = [buf0, buf1]
    gsems = [gsem0, gsem1]
    ssems = [ssem0, ssem1]

    def gather(r):
        return pltpu.make_async_copy(
            x_hbm.at[row0 + r], bufs[r % 2], gsems[r % 2]
        )

    def scatter(r):
        return pltpu.make_async_copy(
            bufs[r % 2], out_hbm.at[row0 + r], ssems[r % 2]
        )

    gather(0).start()
    for r in range(ROWS_PER_WORKER):
        gather(r).wait()
        if r + 1 < ROWS_PER_WORKER:
            if r >= 1:
                scatter(r - 1).wait()
            gather(r + 1).start()
        _row_sparsemax(bufs[r % 2], cbuf)
        scatter(r).start()
    scatter(ROWS_PER_WORKER - 2).wait()
    scatter(ROWS_PER_WORKER - 1).wait()


def kernel(input):
    return _sparsemax_sc(input)
